# Initial kernel scaffold; baseline (speedup 1.0000x reference)
import jax
import jax.numpy as jnp
from jax.experimental import pallas as pl


def kernel(this_level_g, this_level_h, idx):
    raise NotImplementedError("write your pallas kernel here")


# speedup vs baseline: 1.2215x; 1.2215x over previous
import functools

import jax
import jax.numpy as jnp
from jax import lax
from jax.experimental import pallas as pl
from jax.experimental.pallas import tpu as pltpu
from jax.experimental.pallas import tpu_sc as plsc

_NC = 2
_NS = 16
_NW = _NC * _NS
_CHUNK = 392


@functools.partial(jax.jit, static_argnums=(2, 3))
def _sc_gather(table, idx_pad, n_chunks, chunk):
    V, D = table.shape
    b_per_w = n_chunks * chunk
    B_pad = b_per_w * _NW
    mesh = plsc.VectorSubcoreMesh(core_axis_name="c", subcore_axis_name="s")

    @functools.partial(
        pl.kernel,
        mesh=mesh,
        out_type=jax.ShapeDtypeStruct((B_pad, D), jnp.float32),
        scratch_types=[
            pltpu.VMEM((chunk,), jnp.int32),
            pltpu.VMEM((chunk,), jnp.int32),
            pltpu.VMEM((chunk, D), jnp.float32),
            pltpu.VMEM((chunk, D), jnp.float32),
            pltpu.SemaphoreType.DMA,
            pltpu.SemaphoreType.DMA,
            pltpu.SemaphoreType.DMA,
            pltpu.SemaphoreType.DMA,
        ],
    )
    def gather_kernel(table_hbm, idx_hbm, out_hbm,
                      idx_v0, idx_v1, rows_v0, rows_v1,
                      gsem0, gsem1, wsem0, wsem1):
        wid = lax.axis_index("s") * _NC + lax.axis_index("c")
        base = wid * b_per_w
        bufs = ((idx_v0, rows_v0, gsem0, wsem0),
                (idx_v1, rows_v1, gsem1, wsem1))
        gathers = [None] * n_chunks
        writes = [None] * n_chunks
        ib0, rb0, gs0, _ = bufs[0]
        pltpu.sync_copy(idx_hbm.at[pl.ds(base, chunk)], ib0)
        gathers[0] = pltpu.async_copy(table_hbm.at[ib0], rb0, gs0)
        for c in range(n_chunks):
            _, rb, _, ws = bufs[c % 2]
            nxt = c + 1
            if nxt < n_chunks:
                ib2, rb2, gs2, _ = bufs[nxt % 2]
                if nxt >= 2:
                    writes[nxt - 2].wait()
                pltpu.sync_copy(idx_hbm.at[pl.ds(base + nxt * chunk, chunk)], ib2)
                gathers[nxt] = pltpu.async_copy(table_hbm.at[ib2], rb2, gs2)
            gathers[c].wait()
            writes[c] = pltpu.async_copy(
                rb, out_hbm.at[pl.ds(base + c * chunk, chunk)], ws)
        writes[-1].wait()
        if n_chunks >= 2:
            writes[-2].wait()

    return gather_kernel(table, idx_pad)


def kernel(this_level_g, this_level_h, idx):
    del this_level_g
    B = idx.shape[0]
    per_round = _NW * _CHUNK
    n_chunks = -(-B // per_round)
    B_pad = n_chunks * per_round
    idx32 = idx.astype(jnp.int32)
    idx_pad = jnp.pad(idx32, (0, B_pad - B))
    out = _sc_gather(this_level_h, idx_pad, n_chunks, _CHUNK)
    return out[:B]


# speedup vs baseline: 2.0252x; 1.6580x over previous
import functools

import jax
import jax.numpy as jnp
from jax import lax
from jax.experimental import pallas as pl
from jax.experimental.pallas import tpu as pltpu
from jax.experimental.pallas import tpu_sc as plsc

_NC = 2
_NS = 16
_NW = _NC * _NS
_CHUNK = 392


@functools.partial(jax.jit, static_argnums=(2, 3))
def _sc_gather(table, idx, n_chunks, chunk):
    V, D = table.shape
    B = idx.shape[0]
    b_per_w = n_chunks * chunk
    last_start = B - chunk
    mesh = plsc.VectorSubcoreMesh(core_axis_name="c", subcore_axis_name="s")

    @functools.partial(
        pl.kernel,
        mesh=mesh,
        out_type=jax.ShapeDtypeStruct((B, D), jnp.float32),
        scratch_types=[
            pltpu.VMEM((chunk,), jnp.int32),
            pltpu.VMEM((chunk,), jnp.int32),
            pltpu.VMEM((chunk, D), jnp.float32),
            pltpu.VMEM((chunk, D), jnp.float32),
            pltpu.SemaphoreType.DMA,
            pltpu.SemaphoreType.DMA,
            pltpu.SemaphoreType.DMA,
            pltpu.SemaphoreType.DMA,
        ],
    )
    def gather_kernel(table_hbm, idx_hbm, out_hbm,
                      idx_v0, idx_v1, rows_v0, rows_v1,
                      gsem0, gsem1, wsem0, wsem1):
        wid = lax.axis_index("s") * _NC + lax.axis_index("c")
        base = wid * b_per_w
        starts = [jnp.minimum(base + c * chunk, last_start)
                  for c in range(n_chunks)]
        bufs = ((idx_v0, rows_v0, gsem0, wsem0),
                (idx_v1, rows_v1, gsem1, wsem1))
        gathers = [None] * n_chunks
        writes = [None] * n_chunks
        ib0, rb0, gs0, _ = bufs[0]
        pltpu.sync_copy(idx_hbm.at[pl.ds(starts[0], chunk)], ib0)
        gathers[0] = pltpu.async_copy(table_hbm.at[ib0], rb0, gs0)
        for c in range(n_chunks):
            _, rb, _, ws = bufs[c % 2]
            nxt = c + 1
            if nxt < n_chunks:
                ib2, rb2, gs2, _ = bufs[nxt % 2]
                if nxt >= 2:
                    writes[nxt - 2].wait()
                pltpu.sync_copy(idx_hbm.at[pl.ds(starts[nxt], chunk)], ib2)
                gathers[nxt] = pltpu.async_copy(table_hbm.at[ib2], rb2, gs2)
            gathers[c].wait()
            writes[c] = pltpu.async_copy(
                rb, out_hbm.at[pl.ds(starts[c], chunk)], ws)
        writes[-1].wait()
        if n_chunks >= 2:
            writes[-2].wait()

    return gather_kernel(table, idx)


def kernel(this_level_g, this_level_h, idx):
    del this_level_g
    B = idx.shape[0]
    per_round = _NW * _CHUNK
    n_chunks = -(-B // per_round)
    return _sc_gather(this_level_h, idx.astype(jnp.int32), n_chunks, _CHUNK)


# speedup vs baseline: 2.1045x; 1.0392x over previous
import functools

import jax
import jax.numpy as jnp
from jax import lax
from jax.experimental import pallas as pl
from jax.experimental.pallas import tpu as pltpu
from jax.experimental.pallas import tpu_sc as plsc

_NC = 2
_NS = 16
_NW = _NC * _NS
_CHUNK = 224
_NBUF = 4
_LEAD = 3


@functools.partial(jax.jit, static_argnums=(2, 3))
def _sc_gather(table, idx, n_chunks, chunk):
    V, D = table.shape
    B = idx.shape[0]
    b_per_w = n_chunks * chunk
    last_start = B - chunk
    nbuf = min(_NBUF, n_chunks)
    lead = min(_LEAD, nbuf - 1, n_chunks - 1)
    mesh = plsc.VectorSubcoreMesh(core_axis_name="c", subcore_axis_name="s")

    @functools.partial(
        pl.kernel,
        mesh=mesh,
        out_type=jax.ShapeDtypeStruct((B, D), jnp.float32),
        scratch_types=(
            [pltpu.VMEM((b_per_w,), jnp.int32)]
            + [pltpu.VMEM((chunk, D), jnp.float32) for _ in range(nbuf)]
            + [pltpu.SemaphoreType.DMA for _ in range(2 * nbuf)]
        ),
    )
    def gather_kernel(table_hbm, idx_hbm, out_hbm, idx_v, *rest):
        rbufs = rest[:nbuf]
        gsems = rest[nbuf:2 * nbuf]
        wsems = rest[2 * nbuf:]
        wid = lax.axis_index("s") * _NC + lax.axis_index("c")
        wbase = pl.multiple_of(jnp.minimum(wid * b_per_w, B - b_per_w), 8)
        starts = [pl.multiple_of(jnp.minimum(wbase + c * chunk, last_start), 8)
                  for c in range(n_chunks)]
        pltpu.sync_copy(idx_hbm.at[pl.ds(wbase, b_per_w)], idx_v)

        def fire_gather(c):
            return pltpu.async_copy(
                table_hbm.at[idx_v.at[pl.ds(pl.multiple_of(starts[c] - wbase, 8), chunk)]],
                rbufs[c % nbuf], gsems[c % nbuf])

        gathers = [None] * n_chunks
        writes = [None] * n_chunks
        for c in range(lead):
            gathers[c] = fire_gather(c)
        for c in range(n_chunks):
            g = c + lead
            if g < n_chunks:
                if g >= nbuf:
                    writes[g - nbuf].wait()
                gathers[g] = fire_gather(g)
            gathers[c].wait()
            writes[c] = pltpu.async_copy(
                rbufs[c % nbuf], out_hbm.at[pl.ds(starts[c], chunk)],
                wsems[c % nbuf])
        for c in range(max(0, n_chunks - nbuf), n_chunks):
            writes[c].wait()

    return gather_kernel(table, idx)


def kernel(this_level_g, this_level_h, idx):
    del this_level_g
    B = idx.shape[0]
    per_round = _NW * _CHUNK
    n_chunks = -(-B // per_round)
    return _sc_gather(this_level_h, idx.astype(jnp.int32), n_chunks, _CHUNK)
